# Initial kernel scaffold; baseline (speedup 1.0000x reference)
import jax
import jax.numpy as jnp
from jax.experimental import pallas as pl


def kernel(x, edge_index, edge_attr, supernode_indices, params):
    raise NotImplementedError("write your pallas kernel here")


# speedup vs baseline: 29.2589x; 29.2589x over previous
import functools

import jax
import jax.numpy as jnp
from jax import lax
from jax.experimental import pallas as pl
from jax.experimental.pallas import tpu as pltpu
from jax.experimental.pallas import tpu_sc as plsc

F32 = jnp.float32

_NC, _NS, _L = 2, 16, 16
_NW = _NC * _NS
_CH = 1024


def _make_segsum(E, T, n_nodes):
    per = -(-E // _NW // _CH) * _CH
    last = E - per * (_NW - 1)
    assert last > 0 and last % _CH == 0 and per % 128 == 0
    NP = -(-n_nodes // (_NS * 128)) * _NS * 128
    nsub = NP // _NS
    assert nsub % _CH <= _CH and nsub % 16 == 0
    assert NP == (6 * _NS + 2) * _CH

    mesh = plsc.VectorSubcoreMesh(core_axis_name="c", subcore_axis_name="s")

    @functools.partial(
        pl.kernel,
        mesh=mesh,
        compiler_params=pltpu.CompilerParams(use_tc_tiling_on_sc=False),
        out_type=(
            jax.ShapeDtypeStruct((_NC, NP, T), F32),
            jax.ShapeDtypeStruct((_NC, NP), F32),
        ),
        scratch_types=[
            pltpu.VMEM((_CH // 128, 128), jnp.int32),
            pltpu.VMEM((_CH, T), F32),
            pltpu.VMEM((128,), F32),
            pltpu.VMEM((_CH,), F32),
            pltpu.VMEM_SHARED((NP, T), F32),
            pltpu.VMEM_SHARED((NP,), F32),
        ],
    )
    def seg(src_hbm, ea_hbm, a_out, c_out, idx_v, rows_v, ones_v, z1_v,
            acc_sh, cnt_sh):
        cid = lax.axis_index("c")
        sid = lax.axis_index("s")
        wid = sid * _NC + cid

        def _zrow(i, _):
            rows_v[i, :] = jnp.zeros((_L,), F32)
            return 0
        lax.fori_loop(0, _CH, _zrow, 0)
        for i in range(128 // _L):
            ones_v[pl.ds(i * _L, _L)] = jnp.ones((_L,), F32)
        def _zel(i, _):
            z1_v[pl.ds(i * _L, _L)] = jnp.zeros((_L,), F32)
            return 0
        lax.fori_loop(0, _CH // _L, _zel, 0)

        def _zero_rows(base, ln):
            nf, rt = ln // _CH, ln % _CH
            for j in range(nf):
                pltpu.sync_copy(rows_v, acc_sh.at[pl.ds(base + j * _CH, _CH)])
            if rt:
                pltpu.sync_copy(rows_v.at[pl.ds(0, rt)],
                                acc_sh.at[pl.ds(base + nf * _CH, rt)])

        _zero_rows(pl.multiple_of(sid * nsub, 16), nsub)
        for j in range(6):
            k = pl.multiple_of((sid + j * _NS) * _CH, _CH)
            pltpu.sync_copy(z1_v, cnt_sh.at[pl.ds(k, _CH)])

        @pl.when(sid < 2)
        def _():
            k = pl.multiple_of((6 * _NS + sid) * _CH, _CH)
            pltpu.sync_copy(z1_v, cnt_sh.at[pl.ds(k, _CH)])
        plsc.subcore_barrier()

        nch = jnp.where(wid == _NW - 1, last // _CH, per // _CH)

        def chunk(j, _):
            eoff = pl.multiple_of(wid * per + j * _CH, _CH)
            pltpu.sync_copy(src_hbm.at[pl.ds(
                pl.multiple_of(wid * (per // 128) + j * (_CH // 128), 8),
                _CH // 128)], idx_v)
            pltpu.sync_copy(ea_hbm.at[pl.ds(eoff, _CH)], rows_v)
            for sj in range(_CH // 128):
                pltpu.sync_copy(rows_v.at[pl.ds(sj * 128, 128)],
                                acc_sh.at[idx_v.at[sj]], add=True)
                pltpu.sync_copy(ones_v, cnt_sh.at[idx_v.at[sj]], add=True)
            return 0
        lax.fori_loop(0, nch, chunk, 0)
        plsc.subcore_barrier()

        b = pl.multiple_of(sid * nsub, 16)
        pltpu.sync_copy(acc_sh.at[pl.ds(b, nsub)],
                        a_out.at[cid, pl.ds(b, nsub)])
        for j in range(6):
            k = pl.multiple_of((sid + j * _NS) * _CH, _CH)
            pltpu.sync_copy(cnt_sh.at[pl.ds(k, _CH)],
                            c_out.at[cid, pl.ds(k, _CH)])

        @pl.when(sid < 2)
        def _():
            k = pl.multiple_of((6 * _NS + sid) * _CH, _CH)
            pltpu.sync_copy(cnt_sh.at[pl.ds(k, _CH)],
                            c_out.at[cid, pl.ds(k, _CH)])

    return seg


def _tc_layers(x, a_p, c_p, W1s, b1s, W2s, b2s, gs, bes, Wt1, bt1, Wt2, sc3):
    N, T = x.shape
    R = 1024
    grid = (-(-N // R),)
    nl = W1s.shape[0]

    def body(x_ref, a_ref, c_ref, W1_ref, b1_ref, W2_ref, b2_ref, g_ref,
             be_ref, Wt1_ref, bt1_ref, Wt2_ref, sc_ref, t16_ref):
        i = pl.program_id(0)
        A = a_ref[0] + a_ref[1]
        c = c_ref[0, pl.ds(i * R, R)] + c_ref[1, pl.ds(i * R, R)]
        abar = A / jnp.maximum(c, 1.0)[:, None]
        h = x_ref[...]
        for l in range(nl):
            z = jnp.concatenate([h, h * abar], axis=1)
            z = jnp.maximum(z @ W1_ref[l] + b1_ref[l], 0.0)
            z = z @ W2_ref[l] + b2_ref[l]
            mu = jnp.mean(z, axis=1, keepdims=True)
            var = jnp.mean((z - mu) ** 2, axis=1, keepdims=True)
            h = (z - mu) * lax.rsqrt(var + 1e-5) * g_ref[l] + be_ref[l]
        t = jnp.maximum(h @ Wt1_ref[...] + bt1_ref[...], 0.0)
        t = t @ Wt2_ref[...] + sc_ref[0, 0]
        mu_t = t
        var_t = jnp.zeros_like(t)
        tl = (t - mu_t) * lax.rsqrt(var_t + 1e-5) * sc_ref[0, 1] + sc_ref[0, 2]
        t16_ref[...] = jnp.broadcast_to(tl, (R, T))

    full = lambda s: pl.BlockSpec(s, lambda i: tuple(0 for _ in s))
    return pl.pallas_call(
        body,
        grid=grid,
        in_specs=[
            pl.BlockSpec((R, T), lambda i: (i, 0)),
            pl.BlockSpec((_NC, R, T), lambda i: (0, i, 0)),
            pl.BlockSpec(c_p.shape, lambda i: (0, 0)),
            full(W1s.shape), full(b1s.shape), full(W2s.shape),
            full(b2s.shape), full(gs.shape), full(bes.shape),
            full(Wt1.shape), full(bt1.shape), full(Wt2.shape),
            full(sc3.shape),
        ],
        out_specs=pl.BlockSpec((R, T), lambda i: (i, 0)),
        out_shape=jax.ShapeDtypeStruct((N, T), F32),
    )(x, a_p, c_p, W1s, b1s, W2s, b2s, gs, bes, Wt1, bt1, Wt2, sc3)


def _make_tail(n_nodes, T, S, H1, H2):
    assert S % 128 == 0 and H1 % _L == 0 and H2 % _L == 0
    WCH = 256

    mesh = plsc.VectorSubcoreMesh(core_axis_name="c", subcore_axis_name="s")

    @functools.partial(
        pl.kernel,
        mesh=mesh,
        compiler_params=pltpu.CompilerParams(use_tc_tiling_on_sc=False,
                                             needs_layout_passes=False),
        out_type=jax.ShapeDtypeStruct((_L,), F32),
        scratch_types=[
            pltpu.VMEM((S // 128, 128), jnp.int32),
            pltpu.VMEM((S, T), F32),
            pltpu.VMEM((WCH, H1), F32),
            pltpu.VMEM((H1, H2), F32),
            pltpu.VMEM((H1,), F32),
            pltpu.VMEM((H2,), F32),
            pltpu.VMEM((H2,), F32),
            pltpu.VMEM((_L,), F32),
            pltpu.VMEM((_L,), F32),
        ],
    )
    def tail(t16_hbm, sidx_hbm, ws1_hbm, bs1_hbm, ws2_hbm, bs2_hbm, ws3_hbm,
             bs3_hbm, out_hbm, idx_v, sn_v, w1_v, w2_v, acc1_v, acc2_v,
             w3_v, b3_v, out_v):
        cid = lax.axis_index("c")
        sid = lax.axis_index("s")

        @pl.when(jnp.logical_and(cid == 0, sid == 0))
        def _():
            pltpu.sync_copy(sidx_hbm, idx_v)
            for j in range(S // 128):
                pltpu.sync_copy(t16_hbm.at[idx_v.at[j]],
                                sn_v.at[pl.ds(j * 128, 128)])
            pltpu.sync_copy(bs1_hbm, acc1_v)
            for cc in range(S // WCH):
                pltpu.sync_copy(ws1_hbm.at[pl.ds(cc * WCH, WCH)], w1_v)

                def b1(s, _):
                    qv = sn_v[cc * WCH + s, :]
                    for k in range(H1 // _L):
                        plsc.addupdate(acc1_v.at[pl.ds(k * _L, _L)],
                                       qv * w1_v[s, pl.ds(k * _L, _L)])
                    return 0
                lax.fori_loop(0, WCH, b1, 0)
            for k in range(H1 // _L):
                acc1_v[pl.ds(k * _L, _L)] = jnp.maximum(
                    acc1_v[pl.ds(k * _L, _L)], 0.0)
            pltpu.sync_copy(ws2_hbm, w2_v)
            pltpu.sync_copy(bs2_hbm, acc2_v)

            def b2(sb, _):
                vec = acc1_v[pl.ds(sb * _L, _L)]
                for l in range(_L):
                    qs = vec[l]
                    for k in range(H2 // _L):
                        plsc.addupdate(acc2_v.at[pl.ds(k * _L, _L)],
                                       qs * w2_v[sb * _L + l, pl.ds(k * _L, _L)])
                return 0
            lax.fori_loop(0, H1 // _L, b2, 0)
            pltpu.sync_copy(ws3_hbm, w3_v)
            pltpu.sync_copy(bs3_hbm, b3_v)
            vsum = jnp.zeros((_L,), F32)
            for k in range(H2 // _L):
                vsum = vsum + jnp.maximum(acc2_v[pl.ds(k * _L, _L)], 0.0) \
                    * w3_v[pl.ds(k * _L, _L)]
            logit = jnp.sum(vsum) + b3_v[...][0]
            vlogit = jnp.full((_L,), logit, F32)
            m = vlogit
            e = jnp.exp(vlogit - m)
            out_v[...] = e / e
            pltpu.sync_copy(out_v, out_hbm)

    return tail


def kernel(x, edge_index, edge_attr, supernode_indices, params):
    N, T = x.shape
    E = edge_attr.shape[0]
    S = supernode_indices.shape[0]
    lys = params['layers']
    W1s = jnp.stack([p['W1'] for p in lys])
    b1s = jnp.stack([p['b1'] for p in lys])
    W2s = jnp.stack([p['W2'] for p in lys])
    b2s = jnp.stack([p['b2'] for p in lys])
    gs = jnp.stack([p['g'] for p in lys])
    bes = jnp.stack([p['be'] for p in lys])
    Wt1, bt1, Wt2 = params['Wt1'], params['bt1'].reshape(1, -1), params['Wt2']
    sc3 = jnp.stack([params['bt2'][0], params['gt'][0], params['bt'][0]]
                    ).reshape(1, 3)
    H1 = params['Ws1'].shape[1]
    H2 = params['Ws2'].shape[1]

    src2d = edge_index[0].reshape(E // 128, 128)
    a_p, c_p = _make_segsum(E, T, N)(src2d, edge_attr)
    t16 = _tc_layers(x, a_p, c_p, W1s, b1s, W2s, b2s, gs, bes, Wt1, bt1,
                     Wt2, sc3)
    sidx2 = supernode_indices.reshape(S // 128, 128)
    ws3 = params['Ws3'].reshape(-1)
    bs3p = jnp.pad(params['bs3'], (0, _L - params['bs3'].shape[0]))
    out = _make_tail(N, T, S, H1, H2)(
        t16, sidx2, params['Ws1'], params['bs1'], params['Ws2'],
        params['bs2'], ws3, bs3p)
    return out[:1].reshape(1, 1)


# speedup vs baseline: 40.7981x; 1.3944x over previous
import functools

import jax
import jax.numpy as jnp
from jax import lax
from jax.experimental import pallas as pl
from jax.experimental.pallas import tpu as pltpu
from jax.experimental.pallas import tpu_sc as plsc

F32 = jnp.float32

_NC, _NS, _L = 2, 16, 16
_NW = _NC * _NS
_CH = 512


def _make_segsum(E, T, n_nodes):
    per = -(-E // _NW // _CH) * _CH
    last = E - per * (_NW - 1)
    assert last > 0 and last % _CH == 0 and per % 128 == 0
    NP = -(-n_nodes // (_NS * 128)) * _NS * 128
    nsub = NP // _NS
    assert nsub % 16 == 0
    NCK = NP // _CH
    assert NCK * _CH == NP
    crounds, crem = NCK // _NS, NCK % _NS

    mesh = plsc.VectorSubcoreMesh(core_axis_name="c", subcore_axis_name="s")

    @functools.partial(
        pl.kernel,
        mesh=mesh,
        compiler_params=pltpu.CompilerParams(use_tc_tiling_on_sc=False,
                                             needs_layout_passes=False),
        out_type=(
            jax.ShapeDtypeStruct((_NC, NP, T), F32),
            jax.ShapeDtypeStruct((_NC, NP), F32),
        ),
        scratch_types=[
            pltpu.VMEM((_CH // 128, 1, 128), jnp.int32),
            pltpu.VMEM((2, _CH // 128, 8, 128), F32),
            pltpu.VMEM((_CH, T), F32),
            pltpu.VMEM((128,), F32),
            pltpu.VMEM((_CH,), F32),
            pltpu.VMEM_SHARED((NP, T), F32),
            pltpu.VMEM_SHARED((NP,), F32),
        ],
    )
    def seg(src_hbm, ea4_hbm, a_out, c_out, idx_v, buf_v, rows_v, ones_v,
            z1_v, acc_sh, cnt_sh):
        cid = lax.axis_index("c")
        sid = lax.axis_index("s")
        wid = sid * _NC + cid
        iota16 = jnp.arange(_L, dtype=jnp.int32)
        cols = [jnp.full((_L,), c, jnp.int32) for c in range(T)]

        def _zrow(i, _):
            rows_v[i, :] = jnp.zeros((_L,), F32)
            return 0
        lax.fori_loop(0, _CH, _zrow, 0)
        for i in range(128 // _L):
            ones_v[pl.ds(i * _L, _L)] = jnp.ones((_L,), F32)
        def _zel(i, _):
            z1_v[pl.ds(i * _L, _L)] = jnp.zeros((_L,), F32)
            return 0
        lax.fori_loop(0, _CH // _L, _zel, 0)

        def _zero_rows(base, ln):
            nf, rt = ln // _CH, ln % _CH
            for j in range(nf):
                pltpu.sync_copy(rows_v, acc_sh.at[pl.ds(base + j * _CH, _CH)])
            if rt:
                pltpu.sync_copy(rows_v.at[pl.ds(0, rt)],
                                acc_sh.at[pl.ds(base + nf * _CH, rt)])

        _zero_rows(pl.multiple_of(sid * nsub, 16), nsub)
        for j in range(crounds):
            k = pl.multiple_of((sid + j * _NS) * _CH, _CH)
            pltpu.sync_copy(z1_v, cnt_sh.at[pl.ds(k, _CH)])

        @pl.when(sid < crem)
        def _():
            k = pl.multiple_of((crounds * _NS + sid) * _CH, _CH)
            pltpu.sync_copy(z1_v, cnt_sh.at[pl.ds(k, _CH)])
        plsc.subcore_barrier()

        nch = jnp.where(wid == _NW - 1, last // _CH, per // _CH)

        def chunk(j, _):
            eb0 = pl.multiple_of(wid * (per // 128) + j * (_CH // 128),
                                 _CH // 128)
            pltpu.sync_copy(src_hbm.at[pl.ds(eb0, _CH // 128), pl.ds(0, 1)],
                            idx_v)
            pltpu.sync_copy(ea4_hbm.at[0, pl.ds(eb0, _CH // 128)],
                            buf_v.at[0])
            pltpu.sync_copy(ea4_hbm.at[1, pl.ds(eb0, _CH // 128)],
                            buf_v.at[1])

            def trow(ebl, _):
                def tgrp(eg, _):
                    rowidx = (ebl * 128 + eg * _L) + iota16
                    for fb in range(2):
                        for f in range(8):
                            v = buf_v[fb, ebl, f, pl.ds(eg * _L, _L)]
                            plsc.store_scatter(rows_v, [rowidx, cols[fb * 8 + f]], v)
                    return 0
                lax.fori_loop(0, 128 // _L, tgrp, 0)
                return 0
            lax.fori_loop(0, _CH // 128, trow, 0)

            for sj in range(_CH // 128):
                pltpu.sync_copy(rows_v.at[pl.ds(sj * 128, 128)],
                                acc_sh.at[idx_v.at[sj, 0]], add=True)
                pltpu.sync_copy(ones_v, cnt_sh.at[idx_v.at[sj, 0]], add=True)
            return 0
        lax.fori_loop(0, nch, chunk, 0)
        plsc.subcore_barrier()

        b = pl.multiple_of(sid * nsub, 16)
        pltpu.sync_copy(acc_sh.at[pl.ds(b, nsub)],
                        a_out.at[cid, pl.ds(b, nsub)])
        for j in range(crounds):
            k = pl.multiple_of((sid + j * _NS) * _CH, _CH)
            pltpu.sync_copy(cnt_sh.at[pl.ds(k, _CH)],
                            c_out.at[cid, pl.ds(k, _CH)])

        @pl.when(sid < crem)
        def _():
            k = pl.multiple_of((crounds * _NS + sid) * _CH, _CH)
            pltpu.sync_copy(cnt_sh.at[pl.ds(k, _CH)],
                            c_out.at[cid, pl.ds(k, _CH)])

    return seg


def _tc_layers(x, a_p, c_p, W1s, b1s, W2s, b2s, gs, bes, Wt1, bt1, Wt2, sc3):
    N, T = x.shape
    R = 1024
    grid = (-(-N // R),)
    nl = W1s.shape[0]

    def body(x_ref, a_ref, c_ref, W1_ref, b1_ref, W2_ref, b2_ref, g_ref,
             be_ref, Wt1_ref, bt1_ref, Wt2_ref, sc_ref, t16_ref):
        i = pl.program_id(0)
        A = a_ref[0] + a_ref[1]
        c = c_ref[0, pl.ds(i * R, R)] + c_ref[1, pl.ds(i * R, R)]
        abar = A / jnp.maximum(c, 1.0)[:, None]
        h = x_ref[...]
        for l in range(nl):
            z = jnp.concatenate([h, h * abar], axis=1)
            z = jnp.maximum(z @ W1_ref[l] + b1_ref[l], 0.0)
            z = z @ W2_ref[l] + b2_ref[l]
            mu = jnp.mean(z, axis=1, keepdims=True)
            var = jnp.mean((z - mu) ** 2, axis=1, keepdims=True)
            h = (z - mu) * lax.rsqrt(var + 1e-5) * g_ref[l] + be_ref[l]
        t = jnp.maximum(h @ Wt1_ref[...] + bt1_ref[...], 0.0)
        t = t @ Wt2_ref[...] + sc_ref[0, 0]
        mu_t = t
        var_t = jnp.zeros_like(t)
        tl = (t - mu_t) * lax.rsqrt(var_t + 1e-5) * sc_ref[0, 1] + sc_ref[0, 2]
        t16_ref[...] = jnp.broadcast_to(tl, (R, T))

    full = lambda s: pl.BlockSpec(s, lambda i: tuple(0 for _ in s))
    return pl.pallas_call(
        body,
        grid=grid,
        in_specs=[
            pl.BlockSpec((R, T), lambda i: (i, 0)),
            pl.BlockSpec((_NC, R, T), lambda i: (0, i, 0)),
            pl.BlockSpec(c_p.shape, lambda i: (0, 0)),
            full(W1s.shape), full(b1s.shape), full(W2s.shape),
            full(b2s.shape), full(gs.shape), full(bes.shape),
            full(Wt1.shape), full(bt1.shape), full(Wt2.shape),
            full(sc3.shape),
        ],
        out_specs=pl.BlockSpec((R, T), lambda i: (i, 0)),
        out_shape=jax.ShapeDtypeStruct((N, T), F32),
    )(x, a_p, c_p, W1s, b1s, W2s, b2s, gs, bes, Wt1, bt1, Wt2, sc3)


def _make_tail(n_nodes, T, S, H1, H2):
    assert S % 128 == 0 and H1 % _L == 0 and H2 % _L == 0
    WCH = 256

    mesh = plsc.VectorSubcoreMesh(core_axis_name="c", subcore_axis_name="s")

    @functools.partial(
        pl.kernel,
        mesh=mesh,
        compiler_params=pltpu.CompilerParams(use_tc_tiling_on_sc=False,
                                             needs_layout_passes=False),
        out_type=jax.ShapeDtypeStruct((_L,), F32),
        scratch_types=[
            pltpu.VMEM((S // 128, 128), jnp.int32),
            pltpu.VMEM((S, T), F32),
            pltpu.VMEM((WCH, H1), F32),
            pltpu.VMEM((H1, H2), F32),
            pltpu.VMEM((H1,), F32),
            pltpu.VMEM((H2,), F32),
            pltpu.VMEM((H2,), F32),
            pltpu.VMEM((_L,), F32),
            pltpu.VMEM((_L,), F32),
        ],
    )
    def tail(t16_hbm, sidx_hbm, ws1_hbm, bs1_hbm, ws2_hbm, bs2_hbm, ws3_hbm,
             bs3_hbm, out_hbm, idx_v, sn_v, w1_v, w2_v, acc1_v, acc2_v,
             w3_v, b3_v, out_v):
        cid = lax.axis_index("c")
        sid = lax.axis_index("s")

        @pl.when(jnp.logical_and(cid == 0, sid == 0))
        def _():
            pltpu.sync_copy(sidx_hbm, idx_v)
            for j in range(S // 128):
                pltpu.sync_copy(t16_hbm.at[idx_v.at[j]],
                                sn_v.at[pl.ds(j * 128, 128)])
            pltpu.sync_copy(bs1_hbm, acc1_v)
            for cc in range(S // WCH):
                pltpu.sync_copy(ws1_hbm.at[pl.ds(cc * WCH, WCH)], w1_v)

                def b1(s, _):
                    qv = sn_v[cc * WCH + s, :]
                    for k in range(H1 // _L):
                        plsc.addupdate(acc1_v.at[pl.ds(k * _L, _L)],
                                       qv * w1_v[s, pl.ds(k * _L, _L)])
                    return 0
                lax.fori_loop(0, WCH, b1, 0)
            for k in range(H1 // _L):
                acc1_v[pl.ds(k * _L, _L)] = jnp.maximum(
                    acc1_v[pl.ds(k * _L, _L)], 0.0)
            pltpu.sync_copy(ws2_hbm, w2_v)
            pltpu.sync_copy(bs2_hbm, acc2_v)

            def b2(sb, _):
                vec = acc1_v[pl.ds(sb * _L, _L)]
                for l in range(_L):
                    qs = vec[l]
                    for k in range(H2 // _L):
                        plsc.addupdate(acc2_v.at[pl.ds(k * _L, _L)],
                                       qs * w2_v[sb * _L + l, pl.ds(k * _L, _L)])
                return 0
            lax.fori_loop(0, H1 // _L, b2, 0)
            pltpu.sync_copy(ws3_hbm, w3_v)
            pltpu.sync_copy(bs3_hbm, b3_v)
            vsum = jnp.zeros((_L,), F32)
            for k in range(H2 // _L):
                vsum = vsum + jnp.maximum(acc2_v[pl.ds(k * _L, _L)], 0.0) \
                    * w3_v[pl.ds(k * _L, _L)]
            logit = jnp.sum(vsum) + b3_v[...][0]
            vlogit = jnp.full((_L,), logit, F32)
            m = vlogit
            e = jnp.exp(vlogit - m)
            out_v[...] = e / e
            pltpu.sync_copy(out_v, out_hbm)

    return tail


def kernel(x, edge_index, edge_attr, supernode_indices, params):
    N, T = x.shape
    E = edge_attr.shape[0]
    S = supernode_indices.shape[0]
    lys = params['layers']
    W1s = jnp.stack([p['W1'] for p in lys])
    b1s = jnp.stack([p['b1'] for p in lys])
    W2s = jnp.stack([p['W2'] for p in lys])
    b2s = jnp.stack([p['b2'] for p in lys])
    gs = jnp.stack([p['g'] for p in lys])
    bes = jnp.stack([p['be'] for p in lys])
    Wt1, bt1, Wt2 = params['Wt1'], params['bt1'].reshape(1, -1), params['Wt2']
    sc3 = jnp.stack([params['bt2'][0], params['gt'][0], params['bt'][0]]
                    ).reshape(1, 3)
    H1 = params['Ws1'].shape[1]
    H2 = params['Ws2'].shape[1]

    src3 = edge_index.reshape(2, E // 128, 128).transpose(1, 0, 2)
    ea4 = edge_attr.T.reshape(2, 8, E // 128, 128).transpose(0, 2, 1, 3)
    a_p, c_p = _make_segsum(E, T, N)(src3, ea4)
    t16 = _tc_layers(x, a_p, c_p, W1s, b1s, W2s, b2s, gs, bes, Wt1, bt1,
                     Wt2, sc3)
    sidx2 = supernode_indices.reshape(S // 128, 128)
    ws3 = params['Ws3'].reshape(-1)
    bs3p = jnp.pad(params['bs3'], (0, _L - params['bs3'].shape[0]))
    out = _make_tail(N, T, S, H1, H2)(
        t16, sidx2, params['Ws1'], params['bs1'], params['Ws2'],
        params['bs2'], ws3, bs3p)
    return out[:1].reshape(1, 1)


# speedup vs baseline: 54.2837x; 1.3305x over previous
import functools

import jax
import jax.numpy as jnp
from jax import lax
from jax.experimental import pallas as pl
from jax.experimental.pallas import tpu as pltpu
from jax.experimental.pallas import tpu_sc as plsc

F32 = jnp.float32

_NC, _NS, _L = 2, 16, 16
_NW = _NC * _NS
_CH = 128


def _make_segsum(E, T, n_nodes):
    per = -(-E // _NW // _CH) * _CH
    last = E - per * (_NW - 1)
    assert last > 0 and last % _CH == 0 and per % 128 == 0
    assert (per // _CH) % 2 == 0 and (last // _CH) % 2 == 0
    NP = -(-n_nodes // (_NS * 128)) * _NS * 128
    nsub = NP // _NS
    assert nsub % 16 == 0
    NCK = NP // _CH
    assert NCK * _CH == NP
    crounds, crem = NCK // _NS, NCK % _NS

    mesh = plsc.VectorSubcoreMesh(core_axis_name="c", subcore_axis_name="s")

    @functools.partial(
        pl.kernel,
        mesh=mesh,
        compiler_params=pltpu.CompilerParams(use_tc_tiling_on_sc=False,
                                             needs_layout_passes=False),
        out_type=(
            jax.ShapeDtypeStruct((_NC, NP, T), F32),
            jax.ShapeDtypeStruct((_NC, NP), F32),
        ),
        scratch_types=[
            pltpu.VMEM((1, 1, 128), jnp.int32),
            pltpu.VMEM((1, 1, 128), jnp.int32),
            pltpu.VMEM((2, 1, 8, 128), F32),
            pltpu.VMEM((2, 1, 8, 128), F32),
            pltpu.VMEM((_CH, T), F32),
            pltpu.VMEM((128,), F32),
            pltpu.VMEM((_CH,), F32),
            pltpu.VMEM_SHARED((NP, T), F32),
            pltpu.VMEM_SHARED((NP,), F32),
            pltpu.SemaphoreType.DMA,
            pltpu.SemaphoreType.DMA,
            pltpu.SemaphoreType.DMA,
            pltpu.SemaphoreType.DMA,
        ],
    )
    def seg(src_hbm, ea4_hbm, a_out, c_out, idx0_v, idx1_v, buf0_v, buf1_v,
            rows_v, ones_v, z1_v, acc_sh, cnt_sh, sl0, sl1, ss0, ss1):
        cid = lax.axis_index("c")
        sid = lax.axis_index("s")
        wid = sid * _NC + cid
        iota16 = jnp.arange(_L, dtype=jnp.int32)
        cols = [jnp.full((_L,), c, jnp.int32) for c in range(T)]

        def _zrow(i, _):
            rows_v[i, :] = jnp.zeros((_L,), F32)
            return 0
        lax.fori_loop(0, _CH, _zrow, 0)
        for i in range(128 // _L):
            ones_v[pl.ds(i * _L, _L)] = jnp.ones((_L,), F32)
        def _zel(i, _):
            z1_v[pl.ds(i * _L, _L)] = jnp.zeros((_L,), F32)
            return 0
        lax.fori_loop(0, _CH // _L, _zel, 0)

        def _zero_rows(base, ln):
            nf, rt = ln // _CH, ln % _CH
            for j in range(nf):
                pltpu.sync_copy(rows_v, acc_sh.at[pl.ds(base + j * _CH, _CH)])
            if rt:
                pltpu.sync_copy(rows_v.at[pl.ds(0, rt)],
                                acc_sh.at[pl.ds(base + nf * _CH, rt)])

        _zero_rows(pl.multiple_of(sid * nsub, 16), nsub)
        for j in range(crounds):
            k = pl.multiple_of((sid + j * _NS) * _CH, _CH)
            pltpu.sync_copy(z1_v, cnt_sh.at[pl.ds(k, _CH)])

        @pl.when(sid < crem)
        def _():
            k = pl.multiple_of((crounds * _NS + sid) * _CH, _CH)
            pltpu.sync_copy(z1_v, cnt_sh.at[pl.ds(k, _CH)])
        plsc.subcore_barrier()

        nch = jnp.where(wid == _NW - 1, last // _CH, per // _CH)
        slots = ((idx0_v, buf0_v, sl0, ss0), (idx1_v, buf1_v, sl1, ss1))

        def loads(j, idxd, bufd, sem):
            eb0 = wid * (per // 128) + j
            return (
                pltpu.make_async_copy(
                    src_hbm.at[pl.ds(eb0, 1), pl.ds(0, 1)], idxd, sem),
                pltpu.make_async_copy(
                    ea4_hbm.at[0, pl.ds(eb0, 1)], bufd.at[0], sem),
                pltpu.make_async_copy(
                    ea4_hbm.at[1, pl.ds(eb0, 1)], bufd.at[1], sem),
            )

        def scats(idxd, sem):
            return (
                pltpu.make_async_copy(rows_v, acc_sh.at[idxd.at[0, 0]], sem),
                pltpu.make_async_copy(ones_v, cnt_sh.at[idxd.at[0, 0]], sem),
            )

        for c in loads(0, idx0_v, buf0_v, sl0):
            c.start()

        def chunk(j, _):
            def run(sl, ot):
                idxd, bufd, lsem, ssem = slots[sl]
                oidx, obuf, olsem, ossem = slots[ot]
                for c in loads(j, idxd, bufd, lsem):
                    c.wait()

                @pl.when(j >= 1)
                def _():
                    for c in scats(oidx, ossem):
                        c.wait()

                @pl.when(j + 1 < nch)
                def _():
                    for c in loads(j + 1, oidx, obuf, olsem):
                        c.start()

                def tgrp(eg, _):
                    rowidx = eg * _L + iota16
                    for fb in range(2):
                        for f in range(8):
                            v = bufd[fb, 0, f, pl.ds(eg * _L, _L)]
                            plsc.store_scatter(
                                rows_v, [rowidx, cols[fb * 8 + f]], v)
                    return 0
                lax.fori_loop(0, 128 // _L, tgrp, 0)
                for c in scats(idxd, ssem):
                    c.start(add=True)

            @pl.when(j % 2 == 0)
            def _():
                run(0, 1)

            @pl.when(j % 2 == 1)
            def _():
                run(1, 0)
            return 0
        lax.fori_loop(0, nch, chunk, 0)
        for c in scats(idx1_v, ss1):
            c.wait()
        plsc.subcore_barrier()

        b = pl.multiple_of(sid * nsub, 16)
        pltpu.sync_copy(acc_sh.at[pl.ds(b, nsub)],
                        a_out.at[cid, pl.ds(b, nsub)])
        for j in range(crounds):
            k = pl.multiple_of((sid + j * _NS) * _CH, _CH)
            pltpu.sync_copy(cnt_sh.at[pl.ds(k, _CH)],
                            c_out.at[cid, pl.ds(k, _CH)])

        @pl.when(sid < crem)
        def _():
            k = pl.multiple_of((crounds * _NS + sid) * _CH, _CH)
            pltpu.sync_copy(cnt_sh.at[pl.ds(k, _CH)],
                            c_out.at[cid, pl.ds(k, _CH)])

    return seg


def _tc_layers(x, a_p, c_p, W1s, b1s, W2s, b2s, gs, bes, Wt1, bt1, Wt2, sc3):
    N, T = x.shape
    R = 1024
    grid = (-(-N // R),)
    nl = W1s.shape[0]

    def body(x_ref, a_ref, c_ref, W1_ref, b1_ref, W2_ref, b2_ref, g_ref,
             be_ref, Wt1_ref, bt1_ref, Wt2_ref, sc_ref, t16_ref):
        i = pl.program_id(0)
        A = a_ref[0] + a_ref[1]
        c = c_ref[0, pl.ds(i * R, R)] + c_ref[1, pl.ds(i * R, R)]
        abar = A / jnp.maximum(c, 1.0)[:, None]
        h = x_ref[...]
        for l in range(nl):
            z = jnp.concatenate([h, h * abar], axis=1)
            z = jnp.maximum(z @ W1_ref[l] + b1_ref[l], 0.0)
            z = z @ W2_ref[l] + b2_ref[l]
            mu = jnp.mean(z, axis=1, keepdims=True)
            var = jnp.mean((z - mu) ** 2, axis=1, keepdims=True)
            h = (z - mu) * lax.rsqrt(var + 1e-5) * g_ref[l] + be_ref[l]
        t = jnp.maximum(h @ Wt1_ref[...] + bt1_ref[...], 0.0)
        t = t @ Wt2_ref[...] + sc_ref[0, 0]
        mu_t = t
        var_t = jnp.zeros_like(t)
        tl = (t - mu_t) * lax.rsqrt(var_t + 1e-5) * sc_ref[0, 1] + sc_ref[0, 2]
        t16_ref[...] = jnp.broadcast_to(tl, (R, T))

    full = lambda s: pl.BlockSpec(s, lambda i: tuple(0 for _ in s))
    return pl.pallas_call(
        body,
        grid=grid,
        in_specs=[
            pl.BlockSpec((R, T), lambda i: (i, 0)),
            pl.BlockSpec((_NC, R, T), lambda i: (0, i, 0)),
            pl.BlockSpec(c_p.shape, lambda i: (0, 0)),
            full(W1s.shape), full(b1s.shape), full(W2s.shape),
            full(b2s.shape), full(gs.shape), full(bes.shape),
            full(Wt1.shape), full(bt1.shape), full(Wt2.shape),
            full(sc3.shape),
        ],
        out_specs=pl.BlockSpec((R, T), lambda i: (i, 0)),
        out_shape=jax.ShapeDtypeStruct((N, T), F32),
    )(x, a_p, c_p, W1s, b1s, W2s, b2s, gs, bes, Wt1, bt1, Wt2, sc3)


def _make_tail(n_nodes, T, S, H1, H2):
    assert S % 128 == 0 and H1 % _L == 0 and H2 % _L == 0
    WCH = 256

    mesh = plsc.VectorSubcoreMesh(core_axis_name="c", subcore_axis_name="s")

    @functools.partial(
        pl.kernel,
        mesh=mesh,
        compiler_params=pltpu.CompilerParams(use_tc_tiling_on_sc=False,
                                             needs_layout_passes=False),
        out_type=jax.ShapeDtypeStruct((_L,), F32),
        scratch_types=[
            pltpu.VMEM((S // 128, 128), jnp.int32),
            pltpu.VMEM((S, T), F32),
            pltpu.VMEM((WCH, H1), F32),
            pltpu.VMEM((H1, H2), F32),
            pltpu.VMEM((H1,), F32),
            pltpu.VMEM((H2,), F32),
            pltpu.VMEM((H2,), F32),
            pltpu.VMEM((_L,), F32),
            pltpu.VMEM((_L,), F32),
        ],
    )
    def tail(t16_hbm, sidx_hbm, ws1_hbm, bs1_hbm, ws2_hbm, bs2_hbm, ws3_hbm,
             bs3_hbm, out_hbm, idx_v, sn_v, w1_v, w2_v, acc1_v, acc2_v,
             w3_v, b3_v, out_v):
        cid = lax.axis_index("c")
        sid = lax.axis_index("s")

        @pl.when(jnp.logical_and(cid == 0, sid == 0))
        def _():
            pltpu.sync_copy(sidx_hbm, idx_v)
            for j in range(S // 128):
                pltpu.sync_copy(t16_hbm.at[idx_v.at[j]],
                                sn_v.at[pl.ds(j * 128, 128)])
            pltpu.sync_copy(bs1_hbm, acc1_v)
            for cc in range(S // WCH):
                pltpu.sync_copy(ws1_hbm.at[pl.ds(cc * WCH, WCH)], w1_v)

                def b1(s, _):
                    qv = sn_v[cc * WCH + s, :]
                    for k in range(H1 // _L):
                        plsc.addupdate(acc1_v.at[pl.ds(k * _L, _L)],
                                       qv * w1_v[s, pl.ds(k * _L, _L)])
                    return 0
                lax.fori_loop(0, WCH, b1, 0)
            for k in range(H1 // _L):
                acc1_v[pl.ds(k * _L, _L)] = jnp.maximum(
                    acc1_v[pl.ds(k * _L, _L)], 0.0)
            pltpu.sync_copy(ws2_hbm, w2_v)
            pltpu.sync_copy(bs2_hbm, acc2_v)

            def b2(sb, _):
                vec = acc1_v[pl.ds(sb * _L, _L)]
                for l in range(_L):
                    qs = vec[l]
                    for k in range(H2 // _L):
                        plsc.addupdate(acc2_v.at[pl.ds(k * _L, _L)],
                                       qs * w2_v[sb * _L + l, pl.ds(k * _L, _L)])
                return 0
            lax.fori_loop(0, H1 // _L, b2, 0)
            pltpu.sync_copy(ws3_hbm, w3_v)
            pltpu.sync_copy(bs3_hbm, b3_v)
            vsum = jnp.zeros((_L,), F32)
            for k in range(H2 // _L):
                vsum = vsum + jnp.maximum(acc2_v[pl.ds(k * _L, _L)], 0.0) \
                    * w3_v[pl.ds(k * _L, _L)]
            logit = jnp.sum(vsum) + b3_v[...][0]
            vlogit = jnp.full((_L,), logit, F32)
            m = vlogit
            e = jnp.exp(vlogit - m)
            out_v[...] = e / e
            pltpu.sync_copy(out_v, out_hbm)

    return tail


def kernel(x, edge_index, edge_attr, supernode_indices, params):
    N, T = x.shape
    E = edge_attr.shape[0]
    S = supernode_indices.shape[0]
    lys = params['layers']
    W1s = jnp.stack([p['W1'] for p in lys])
    b1s = jnp.stack([p['b1'] for p in lys])
    W2s = jnp.stack([p['W2'] for p in lys])
    b2s = jnp.stack([p['b2'] for p in lys])
    gs = jnp.stack([p['g'] for p in lys])
    bes = jnp.stack([p['be'] for p in lys])
    Wt1, bt1, Wt2 = params['Wt1'], params['bt1'].reshape(1, -1), params['Wt2']
    sc3 = jnp.stack([params['bt2'][0], params['gt'][0], params['bt'][0]]
                    ).reshape(1, 3)
    H1 = params['Ws1'].shape[1]
    H2 = params['Ws2'].shape[1]

    src3 = edge_index.reshape(2, E // 128, 128).transpose(1, 0, 2)
    ea4 = edge_attr.T.reshape(2, 8, E // 128, 128).transpose(0, 2, 1, 3)
    a_p, c_p = _make_segsum(E, T, N)(src3, ea4)
    t16 = _tc_layers(x, a_p, c_p, W1s, b1s, W2s, b2s, gs, bes, Wt1, bt1,
                     Wt2, sc3)
    sidx2 = supernode_indices.reshape(S // 128, 128)
    ws3 = params['Ws3'].reshape(-1)
    bs3p = jnp.pad(params['bs3'], (0, _L - params['bs3'].shape[0]))
    out = _make_tail(N, T, S, H1, H2)(
        t16, sidx2, params['Ws1'], params['bs1'], params['Ws2'],
        params['bs2'], ws3, bs3p)
    return out[:1].reshape(1, 1)


# speedup vs baseline: 62.0258x; 1.1426x over previous
import functools

import jax
import jax.numpy as jnp
from jax import lax
from jax.experimental import pallas as pl
from jax.experimental.pallas import tpu as pltpu
from jax.experimental.pallas import tpu_sc as plsc

F32 = jnp.float32

_NC, _NS, _L = 2, 16, 16
_NW = _NC * _NS
_CH = 128


def _make_segsum(E, T, n_nodes):
    per = -(-E // _NW // _CH) * _CH
    last = E - per * (_NW - 1)
    assert last > 0 and last % _CH == 0 and per % 128 == 0
    assert (per // _CH) % 2 == 0 and (last // _CH) % 2 == 0
    NP = -(-n_nodes // (_NS * 128)) * _NS * 128
    nsub = NP // _NS
    assert nsub % 16 == 0
    NCK = NP // _CH
    assert NCK * _CH == NP
    crounds, crem = NCK // _NS, NCK % _NS

    mesh = plsc.VectorSubcoreMesh(core_axis_name="c", subcore_axis_name="s")

    @functools.partial(
        pl.kernel,
        mesh=mesh,
        compiler_params=pltpu.CompilerParams(use_tc_tiling_on_sc=False,
                                             needs_layout_passes=False),
        out_type=(
            jax.ShapeDtypeStruct((_NC, NP, T), F32),
            jax.ShapeDtypeStruct((_NC, NP), F32),
        ),
        scratch_types=[
            pltpu.VMEM((1, 1, 128), jnp.int32),
            pltpu.VMEM((1, 1, 128), jnp.int32),
            pltpu.VMEM((2, 1, 8, 128), F32),
            pltpu.VMEM((2, 1, 8, 128), F32),
            pltpu.VMEM((_CH, T), F32),
            pltpu.VMEM((128,), F32),
            pltpu.VMEM((_CH,), F32),
            pltpu.VMEM_SHARED((NP, T), F32),
            pltpu.VMEM_SHARED((NP,), F32),
            pltpu.SemaphoreType.DMA,
            pltpu.SemaphoreType.DMA,
            pltpu.SemaphoreType.DMA,
            pltpu.SemaphoreType.DMA,
        ],
    )
    def seg(src_hbm, ea4_hbm, a_out, c_out, idx0_v, idx1_v, buf0_v, buf1_v,
            rows_v, ones_v, z1_v, acc_sh, cnt_sh, sl0, sl1, ss0, ss1):
        cid = lax.axis_index("c")
        sid = lax.axis_index("s")
        wid = sid * _NC + cid
        iota16 = jnp.arange(_L, dtype=jnp.int32)
        cols = [jnp.full((_L,), c, jnp.int32) for c in range(T)]

        def _zrow(i, _):
            rows_v[i, :] = jnp.zeros((_L,), F32)
            return 0
        lax.fori_loop(0, _CH, _zrow, 0)
        for i in range(128 // _L):
            ones_v[pl.ds(i * _L, _L)] = jnp.ones((_L,), F32)
        def _zel(i, _):
            z1_v[pl.ds(i * _L, _L)] = jnp.zeros((_L,), F32)
            return 0
        lax.fori_loop(0, _CH // _L, _zel, 0)

        def _zero_rows(base, ln):
            nf, rt = ln // _CH, ln % _CH
            for j in range(nf):
                pltpu.sync_copy(rows_v, acc_sh.at[pl.ds(base + j * _CH, _CH)])
            if rt:
                pltpu.sync_copy(rows_v.at[pl.ds(0, rt)],
                                acc_sh.at[pl.ds(base + nf * _CH, rt)])

        _zero_rows(pl.multiple_of(sid * nsub, 16), nsub)
        for j in range(crounds):
            k = pl.multiple_of((sid + j * _NS) * _CH, _CH)
            pltpu.sync_copy(z1_v, cnt_sh.at[pl.ds(k, _CH)])

        @pl.when(sid < crem)
        def _():
            k = pl.multiple_of((crounds * _NS + sid) * _CH, _CH)
            pltpu.sync_copy(z1_v, cnt_sh.at[pl.ds(k, _CH)])
        plsc.subcore_barrier()

        nch = jnp.where(wid == _NW - 1, last // _CH, per // _CH)
        slots = ((idx0_v, buf0_v, sl0, ss0), (idx1_v, buf1_v, sl1, ss1))

        def loads(j, idxd, bufd, sem):
            eb0 = wid * (per // 128) + j
            return (
                pltpu.make_async_copy(
                    src_hbm.at[pl.ds(eb0, 1), pl.ds(0, 1)], idxd, sem),
                pltpu.make_async_copy(
                    ea4_hbm.at[0, pl.ds(eb0, 1)], bufd.at[0], sem),
                pltpu.make_async_copy(
                    ea4_hbm.at[1, pl.ds(eb0, 1)], bufd.at[1], sem),
            )

        def scats(idxd, sem):
            return (
                pltpu.make_async_copy(rows_v, acc_sh.at[idxd.at[0, 0]], sem),
                pltpu.make_async_copy(ones_v, cnt_sh.at[idxd.at[0, 0]], sem),
            )

        for c in loads(0, idx0_v, buf0_v, sl0):
            c.start()

        def chunk(j, _):
            def run(sl, ot):
                idxd, bufd, lsem, ssem = slots[sl]
                oidx, obuf, olsem, ossem = slots[ot]
                for c in loads(j, idxd, bufd, lsem):
                    c.wait()

                @pl.when(j >= 1)
                def _():
                    for c in scats(oidx, ossem):
                        c.wait()

                @pl.when(j + 1 < nch)
                def _():
                    for c in loads(j + 1, oidx, obuf, olsem):
                        c.start()

                def tgrp(eg, _):
                    rowidx = eg * _L + iota16
                    for fb in range(2):
                        for f in range(8):
                            v = bufd[fb, 0, f, pl.ds(eg * _L, _L)]
                            plsc.store_scatter(
                                rows_v, [rowidx, cols[fb * 8 + f]], v)
                    return 0
                lax.fori_loop(0, 128 // _L, tgrp, 0, unroll=True)
                for c in scats(idxd, ssem):
                    c.start(add=True)

            @pl.when(j % 2 == 0)
            def _():
                run(0, 1)

            @pl.when(j % 2 == 1)
            def _():
                run(1, 0)
            return 0
        lax.fori_loop(0, nch, chunk, 0)
        for c in scats(idx1_v, ss1):
            c.wait()
        plsc.subcore_barrier()

        b = pl.multiple_of(sid * nsub, 16)
        pltpu.sync_copy(acc_sh.at[pl.ds(b, nsub)],
                        a_out.at[cid, pl.ds(b, nsub)])
        for j in range(crounds):
            k = pl.multiple_of((sid + j * _NS) * _CH, _CH)
            pltpu.sync_copy(cnt_sh.at[pl.ds(k, _CH)],
                            c_out.at[cid, pl.ds(k, _CH)])

        @pl.when(sid < crem)
        def _():
            k = pl.multiple_of((crounds * _NS + sid) * _CH, _CH)
            pltpu.sync_copy(cnt_sh.at[pl.ds(k, _CH)],
                            c_out.at[cid, pl.ds(k, _CH)])

    return seg


def _tc_layers(x, a_p, c_p, W1s, b1s, W2s, b2s, gs, bes, Wt1, bt1, Wt2, sc3):
    N, T = x.shape
    R = 2048
    grid = (-(-N // R),)
    nl = W1s.shape[0]

    def body(x_ref, a_ref, c_ref, W1_ref, b1_ref, W2_ref, b2_ref, g_ref,
             be_ref, Wt1_ref, bt1_ref, Wt2_ref, sc_ref, t16_ref):
        i = pl.program_id(0)
        A = a_ref[0] + a_ref[1]
        c = c_ref[0, pl.ds(i * R, R)] + c_ref[1, pl.ds(i * R, R)]
        abar = A / jnp.maximum(c, 1.0)[:, None]
        h = x_ref[...]
        for l in range(nl):
            z = jnp.concatenate([h, h * abar], axis=1)
            z = jnp.maximum(z @ W1_ref[l] + b1_ref[l], 0.0)
            z = z @ W2_ref[l] + b2_ref[l]
            mu = jnp.mean(z, axis=1, keepdims=True)
            var = jnp.mean((z - mu) ** 2, axis=1, keepdims=True)
            h = (z - mu) * lax.rsqrt(var + 1e-5) * g_ref[l] + be_ref[l]
        t = jnp.maximum(h @ Wt1_ref[...] + bt1_ref[...], 0.0)
        t = t @ Wt2_ref[...] + sc_ref[0, 0]
        mu_t = t
        var_t = jnp.zeros_like(t)
        tl = (t - mu_t) * lax.rsqrt(var_t + 1e-5) * sc_ref[0, 1] + sc_ref[0, 2]
        t16_ref[...] = jnp.broadcast_to(tl, (R, T))

    full = lambda s: pl.BlockSpec(s, lambda i: tuple(0 for _ in s))
    return pl.pallas_call(
        body,
        grid=grid,
        in_specs=[
            pl.BlockSpec((R, T), lambda i: (i, 0)),
            pl.BlockSpec((_NC, R, T), lambda i: (0, i, 0)),
            pl.BlockSpec(c_p.shape, lambda i: (0, 0)),
            full(W1s.shape), full(b1s.shape), full(W2s.shape),
            full(b2s.shape), full(gs.shape), full(bes.shape),
            full(Wt1.shape), full(bt1.shape), full(Wt2.shape),
            full(sc3.shape),
        ],
        out_specs=pl.BlockSpec((R, T), lambda i: (i, 0)),
        out_shape=jax.ShapeDtypeStruct((N, T), F32),
    )(x, a_p, c_p, W1s, b1s, W2s, b2s, gs, bes, Wt1, bt1, Wt2, sc3)


def _make_tail(n_nodes, T, S, H1, H2):
    assert S % 128 == 0 and H1 % _L == 0 and H2 % _L == 0
    WCH = 256

    mesh = plsc.VectorSubcoreMesh(core_axis_name="c", subcore_axis_name="s")

    @functools.partial(
        pl.kernel,
        mesh=mesh,
        compiler_params=pltpu.CompilerParams(use_tc_tiling_on_sc=False,
                                             needs_layout_passes=False),
        out_type=jax.ShapeDtypeStruct((_L,), F32),
        scratch_types=[
            pltpu.VMEM((S // 128, 128), jnp.int32),
            pltpu.VMEM((S, T), F32),
            pltpu.VMEM((WCH, H1), F32),
            pltpu.VMEM((H1, H2), F32),
            pltpu.VMEM((H1,), F32),
            pltpu.VMEM((H2,), F32),
            pltpu.VMEM((H2,), F32),
            pltpu.VMEM((_L,), F32),
            pltpu.VMEM((_L,), F32),
        ],
    )
    def tail(t16_hbm, sidx_hbm, ws1_hbm, bs1_hbm, ws2_hbm, bs2_hbm, ws3_hbm,
             bs3_hbm, out_hbm, idx_v, sn_v, w1_v, w2_v, acc1_v, acc2_v,
             w3_v, b3_v, out_v):
        cid = lax.axis_index("c")
        sid = lax.axis_index("s")

        @pl.when(jnp.logical_and(cid == 0, sid == 0))
        def _():
            pltpu.sync_copy(sidx_hbm, idx_v)
            for j in range(S // 128):
                pltpu.sync_copy(t16_hbm.at[idx_v.at[j]],
                                sn_v.at[pl.ds(j * 128, 128)])
            pltpu.sync_copy(bs1_hbm, acc1_v)
            acc = tuple(acc1_v[pl.ds(k * _L, _L)] for k in range(H1 // _L))
            for cc in range(S // WCH):
                pltpu.sync_copy(ws1_hbm.at[pl.ds(cc * WCH, WCH)], w1_v)

                def b1(s, a):
                    qv = sn_v[cc * WCH + s, :]
                    return tuple(a[k] + qv * w1_v[s, pl.ds(k * _L, _L)]
                                 for k in range(H1 // _L))
                acc = lax.fori_loop(0, WCH, b1, acc)
            for k in range(H1 // _L):
                acc1_v[pl.ds(k * _L, _L)] = jnp.maximum(acc[k], 0.0)
            pltpu.sync_copy(ws2_hbm, w2_v)
            pltpu.sync_copy(bs2_hbm, acc2_v)
            acc2 = tuple(acc2_v[pl.ds(k * _L, _L)] for k in range(H2 // _L))

            def b2(sb, a2):
                vec = acc1_v[pl.ds(sb * _L, _L)]
                for l in range(_L):
                    qs = vec[l]
                    a2 = tuple(a2[k] + qs * w2_v[sb * _L + l, pl.ds(k * _L, _L)]
                               for k in range(H2 // _L))
                return a2
            acc2 = lax.fori_loop(0, H1 // _L, b2, acc2)
            pltpu.sync_copy(ws3_hbm, w3_v)
            pltpu.sync_copy(bs3_hbm, b3_v)
            vsum = jnp.zeros((_L,), F32)
            for k in range(H2 // _L):
                vsum = vsum + jnp.maximum(acc2[k], 0.0) \
                    * w3_v[pl.ds(k * _L, _L)]
            logit = jnp.sum(vsum) + b3_v[...][0]
            vlogit = jnp.full((_L,), logit, F32)
            m = vlogit
            e = jnp.exp(vlogit - m)
            out_v[...] = e / e
            pltpu.sync_copy(out_v, out_hbm)

    return tail


def kernel(x, edge_index, edge_attr, supernode_indices, params):
    N, T = x.shape
    E = edge_attr.shape[0]
    S = supernode_indices.shape[0]
    lys = params['layers']
    W1s = jnp.stack([p['W1'] for p in lys])
    b1s = jnp.stack([p['b1'] for p in lys])
    W2s = jnp.stack([p['W2'] for p in lys])
    b2s = jnp.stack([p['b2'] for p in lys])
    gs = jnp.stack([p['g'] for p in lys])
    bes = jnp.stack([p['be'] for p in lys])
    Wt1, bt1, Wt2 = params['Wt1'], params['bt1'].reshape(1, -1), params['Wt2']
    sc3 = jnp.stack([params['bt2'][0], params['gt'][0], params['bt'][0]]
                    ).reshape(1, 3)
    H1 = params['Ws1'].shape[1]
    H2 = params['Ws2'].shape[1]

    src3 = edge_index.reshape(2, E // 128, 128).transpose(1, 0, 2)
    ea4 = edge_attr.T.reshape(2, 8, E // 128, 128).transpose(0, 2, 1, 3)
    a_p, c_p = _make_segsum(E, T, N)(src3, ea4)
    t16 = _tc_layers(x, a_p, c_p, W1s, b1s, W2s, b2s, gs, bes, Wt1, bt1,
                     Wt2, sc3)
    sidx2 = supernode_indices.reshape(S // 128, 128)
    ws3 = params['Ws3'].reshape(-1)
    bs3p = jnp.pad(params['bs3'], (0, _L - params['bs3'].shape[0]))
    out = _make_tail(N, T, S, H1, H2)(
        t16, sidx2, params['Ws1'], params['bs1'], params['Ws2'],
        params['bs2'], ws3, bs3p)
    return out[:1].reshape(1, 1)


# speedup vs baseline: 66.0190x; 1.0644x over previous
import functools

import jax
import jax.numpy as jnp
from jax import lax
from jax.experimental import pallas as pl
from jax.experimental.pallas import tpu as pltpu
from jax.experimental.pallas import tpu_sc as plsc

F32 = jnp.float32

_NC, _NS, _L = 2, 16, 16
_NW = _NC * _NS
_CH = 256


def _make_segsum(E, T, n_nodes):
    per = -(-E // _NW // _CH) * _CH
    last = E - per * (_NW - 1)
    assert last > 0 and last % _CH == 0 and per % 128 == 0
    _SB = _CH // 128
    NP = -(-n_nodes // (_NS * 128)) * _NS * 128
    nsub = NP // _NS
    assert nsub % 16 == 0
    NCK = NP // _CH
    assert NCK * _CH == NP
    crounds, crem = NCK // _NS, NCK % _NS

    mesh = plsc.VectorSubcoreMesh(core_axis_name="c", subcore_axis_name="s")

    @functools.partial(
        pl.kernel,
        mesh=mesh,
        compiler_params=pltpu.CompilerParams(use_tc_tiling_on_sc=False,
                                             needs_layout_passes=False),
        out_type=(
            jax.ShapeDtypeStruct((_NC, NP, T), F32),
            jax.ShapeDtypeStruct((_NC, NP), F32),
        ),
        scratch_types=[
            pltpu.VMEM((_SB, 1, 128), jnp.int32),
            pltpu.VMEM((_SB, 1, 128), jnp.int32),
            pltpu.VMEM((2, _SB, 8, 128), F32),
            pltpu.VMEM((2, _SB, 8, 128), F32),
            pltpu.VMEM((_CH, T), F32),
            pltpu.VMEM((128,), F32),
            pltpu.VMEM((_CH,), F32),
            pltpu.VMEM_SHARED((NP, T), F32),
            pltpu.VMEM_SHARED((NP,), F32),
            pltpu.SemaphoreType.DMA,
            pltpu.SemaphoreType.DMA,
            pltpu.SemaphoreType.DMA,
            pltpu.SemaphoreType.DMA,
        ],
    )
    def seg(src_hbm, ea4_hbm, a_out, c_out, idx0_v, idx1_v, buf0_v, buf1_v,
            rows_v, ones_v, z1_v, acc_sh, cnt_sh, sl0, sl1, ss0, ss1):
        cid = lax.axis_index("c")
        sid = lax.axis_index("s")
        wid = sid * _NC + cid
        iota16 = jnp.arange(_L, dtype=jnp.int32)
        cols = [jnp.full((_L,), c, jnp.int32) for c in range(T)]

        def _zrow(i, _):
            rows_v[i, :] = jnp.zeros((_L,), F32)
            return 0
        lax.fori_loop(0, _CH, _zrow, 0)
        for i in range(128 // _L):
            ones_v[pl.ds(i * _L, _L)] = jnp.ones((_L,), F32)
        def _zel(i, _):
            z1_v[pl.ds(i * _L, _L)] = jnp.zeros((_L,), F32)
            return 0
        lax.fori_loop(0, _CH // _L, _zel, 0)

        def _zero_rows(base, ln):
            nf, rt = ln // _CH, ln % _CH
            for j in range(nf):
                pltpu.sync_copy(rows_v, acc_sh.at[pl.ds(base + j * _CH, _CH)])
            if rt:
                pltpu.sync_copy(rows_v.at[pl.ds(0, rt)],
                                acc_sh.at[pl.ds(base + nf * _CH, rt)])

        _zero_rows(pl.multiple_of(sid * nsub, 16), nsub)
        for j in range(crounds):
            k = pl.multiple_of((sid + j * _NS) * _CH, _CH)
            pltpu.sync_copy(z1_v, cnt_sh.at[pl.ds(k, _CH)])

        @pl.when(sid < crem)
        def _():
            k = pl.multiple_of((crounds * _NS + sid) * _CH, _CH)
            pltpu.sync_copy(z1_v, cnt_sh.at[pl.ds(k, _CH)])
        plsc.subcore_barrier()

        nch = jnp.where(wid == _NW - 1, last // _CH, per // _CH)
        slots = ((idx0_v, buf0_v, sl0, ss0), (idx1_v, buf1_v, sl1, ss1))

        def loads(j, idxd, bufd, sem):
            eb0 = wid * (per // 128) + j * _SB
            return (
                pltpu.make_async_copy(
                    src_hbm.at[pl.ds(eb0, _SB), pl.ds(0, 1)], idxd, sem),
                pltpu.make_async_copy(
                    ea4_hbm.at[0, pl.ds(eb0, _SB)], bufd.at[0], sem),
                pltpu.make_async_copy(
                    ea4_hbm.at[1, pl.ds(eb0, _SB)], bufd.at[1], sem),
            )

        def scats(idxd, sem):
            cps = []
            for sj in range(_SB):
                cps.append(pltpu.make_async_copy(
                    rows_v.at[pl.ds(sj * 128, 128)],
                    acc_sh.at[idxd.at[sj, 0]], sem))
                cps.append(pltpu.make_async_copy(
                    ones_v, cnt_sh.at[idxd.at[sj, 0]], sem))
            return cps

        for c in loads(0, idx0_v, buf0_v, sl0):
            c.start()

        def chunk(j, _):
            def run(sl, ot):
                idxd, bufd, lsem, ssem = slots[sl]
                oidx, obuf, olsem, ossem = slots[ot]
                for c in loads(j, idxd, bufd, lsem):
                    c.wait()

                @pl.when(j >= 1)
                def _():
                    for c in scats(oidx, ossem):
                        c.wait()

                @pl.when(j + 1 < nch)
                def _():
                    for c in loads(j + 1, oidx, obuf, olsem):
                        c.start()

                def tgrp(eg, _):
                    for ebl in range(_SB):
                        rowidx = ebl * 128 + eg * _L + iota16
                        for fb in range(2):
                            for f in range(8):
                                v = bufd[fb, ebl, f, pl.ds(eg * _L, _L)]
                                plsc.store_scatter(
                                    rows_v, [rowidx, cols[fb * 8 + f]], v)
                    return 0
                lax.fori_loop(0, 128 // _L, tgrp, 0, unroll=True)
                for c in scats(idxd, ssem):
                    c.start(add=True)

            @pl.when(j % 2 == 0)
            def _():
                run(0, 1)

            @pl.when(j % 2 == 1)
            def _():
                run(1, 0)
            return 0
        lax.fori_loop(0, nch, chunk, 0)

        @pl.when((nch - 1) % 2 == 0)
        def _():
            for c in scats(idx0_v, ss0):
                c.wait()

        @pl.when((nch - 1) % 2 == 1)
        def _():
            for c in scats(idx1_v, ss1):
                c.wait()
        plsc.subcore_barrier()

        b = pl.multiple_of(sid * nsub, 16)
        pltpu.sync_copy(acc_sh.at[pl.ds(b, nsub)],
                        a_out.at[cid, pl.ds(b, nsub)])
        for j in range(crounds):
            k = pl.multiple_of((sid + j * _NS) * _CH, _CH)
            pltpu.sync_copy(cnt_sh.at[pl.ds(k, _CH)],
                            c_out.at[cid, pl.ds(k, _CH)])

        @pl.when(sid < crem)
        def _():
            k = pl.multiple_of((crounds * _NS + sid) * _CH, _CH)
            pltpu.sync_copy(cnt_sh.at[pl.ds(k, _CH)],
                            c_out.at[cid, pl.ds(k, _CH)])

    return seg


def _tc_layers(x, a_p, c_p, W1s, b1s, W2s, b2s, gs, bes, Wt1, bt1, Wt2, sc3):
    N, T = x.shape
    R = 2048
    grid = (-(-N // R),)
    nl = W1s.shape[0]

    def body(x_ref, a_ref, c_ref, W1_ref, b1_ref, W2_ref, b2_ref, g_ref,
             be_ref, Wt1_ref, bt1_ref, Wt2_ref, sc_ref, t16_ref):
        i = pl.program_id(0)
        A = a_ref[0] + a_ref[1]
        c = c_ref[0, pl.ds(i * R, R)] + c_ref[1, pl.ds(i * R, R)]
        abar = A / jnp.maximum(c, 1.0)[:, None]
        h = x_ref[...]
        for l in range(nl):
            z = jnp.concatenate([h, h * abar], axis=1)
            z = jnp.maximum(z @ W1_ref[l] + b1_ref[l], 0.0)
            z = z @ W2_ref[l] + b2_ref[l]
            mu = jnp.mean(z, axis=1, keepdims=True)
            var = jnp.mean((z - mu) ** 2, axis=1, keepdims=True)
            h = (z - mu) * lax.rsqrt(var + 1e-5) * g_ref[l] + be_ref[l]
        t = jnp.maximum(h @ Wt1_ref[...] + bt1_ref[...], 0.0)
        t = t @ Wt2_ref[...] + sc_ref[0, 0]
        mu_t = t
        var_t = jnp.zeros_like(t)
        tl = (t - mu_t) * lax.rsqrt(var_t + 1e-5) * sc_ref[0, 1] + sc_ref[0, 2]
        t16_ref[...] = jnp.broadcast_to(tl, (R, T))

    full = lambda s: pl.BlockSpec(s, lambda i: tuple(0 for _ in s))
    return pl.pallas_call(
        body,
        grid=grid,
        in_specs=[
            pl.BlockSpec((R, T), lambda i: (i, 0)),
            pl.BlockSpec((_NC, R, T), lambda i: (0, i, 0)),
            pl.BlockSpec(c_p.shape, lambda i: (0, 0)),
            full(W1s.shape), full(b1s.shape), full(W2s.shape),
            full(b2s.shape), full(gs.shape), full(bes.shape),
            full(Wt1.shape), full(bt1.shape), full(Wt2.shape),
            full(sc3.shape),
        ],
        out_specs=pl.BlockSpec((R, T), lambda i: (i, 0)),
        out_shape=jax.ShapeDtypeStruct((N, T), F32),
    )(x, a_p, c_p, W1s, b1s, W2s, b2s, gs, bes, Wt1, bt1, Wt2, sc3)


def _make_tail(n_nodes, T, S, H1, H2):
    assert S % 128 == 0 and H1 % _L == 0 and H2 % _L == 0
    WCH = 256

    mesh = plsc.VectorSubcoreMesh(core_axis_name="c", subcore_axis_name="s")

    @functools.partial(
        pl.kernel,
        mesh=mesh,
        compiler_params=pltpu.CompilerParams(use_tc_tiling_on_sc=False,
                                             needs_layout_passes=False),
        out_type=jax.ShapeDtypeStruct((_L,), F32),
        scratch_types=[
            pltpu.VMEM((S // 128, 128), jnp.int32),
            pltpu.VMEM((S, T), F32),
            pltpu.VMEM((WCH, H1), F32),
            pltpu.VMEM((H1, H2), F32),
            pltpu.VMEM((H1,), F32),
            pltpu.VMEM((H2,), F32),
            pltpu.VMEM((H2,), F32),
            pltpu.VMEM((_L,), F32),
            pltpu.VMEM((_L,), F32),
        ],
    )
    def tail(t16_hbm, sidx_hbm, ws1_hbm, bs1_hbm, ws2_hbm, bs2_hbm, ws3_hbm,
             bs3_hbm, out_hbm, idx_v, sn_v, w1_v, w2_v, acc1_v, acc2_v,
             w3_v, b3_v, out_v):
        cid = lax.axis_index("c")
        sid = lax.axis_index("s")

        @pl.when(jnp.logical_and(cid == 0, sid == 0))
        def _():
            pltpu.sync_copy(sidx_hbm, idx_v)
            for j in range(S // 128):
                pltpu.sync_copy(t16_hbm.at[idx_v.at[j]],
                                sn_v.at[pl.ds(j * 128, 128)])
            pltpu.sync_copy(bs1_hbm, acc1_v)
            acc = tuple(acc1_v[pl.ds(k * _L, _L)] for k in range(H1 // _L))
            for cc in range(S // WCH):
                pltpu.sync_copy(ws1_hbm.at[pl.ds(cc * WCH, WCH)], w1_v)

                def b1(s, a):
                    qv = sn_v[cc * WCH + s, :]
                    return tuple(a[k] + qv * w1_v[s, pl.ds(k * _L, _L)]
                                 for k in range(H1 // _L))
                acc = lax.fori_loop(0, WCH, b1, acc)
            for k in range(H1 // _L):
                acc1_v[pl.ds(k * _L, _L)] = jnp.maximum(acc[k], 0.0)
            pltpu.sync_copy(ws2_hbm, w2_v)
            pltpu.sync_copy(bs2_hbm, acc2_v)
            acc2 = tuple(acc2_v[pl.ds(k * _L, _L)] for k in range(H2 // _L))

            def b2(sb, a2):
                vec = acc1_v[pl.ds(sb * _L, _L)]
                for l in range(_L):
                    qs = vec[l]
                    a2 = tuple(a2[k] + qs * w2_v[sb * _L + l, pl.ds(k * _L, _L)]
                               for k in range(H2 // _L))
                return a2
            acc2 = lax.fori_loop(0, H1 // _L, b2, acc2)
            pltpu.sync_copy(ws3_hbm, w3_v)
            pltpu.sync_copy(bs3_hbm, b3_v)
            vsum = jnp.zeros((_L,), F32)
            for k in range(H2 // _L):
                vsum = vsum + jnp.maximum(acc2[k], 0.0) \
                    * w3_v[pl.ds(k * _L, _L)]
            logit = jnp.sum(vsum) + b3_v[...][0]
            vlogit = jnp.full((_L,), logit, F32)
            m = vlogit
            e = jnp.exp(vlogit - m)
            out_v[...] = e / e
            pltpu.sync_copy(out_v, out_hbm)

    return tail


def kernel(x, edge_index, edge_attr, supernode_indices, params):
    N, T = x.shape
    E = edge_attr.shape[0]
    S = supernode_indices.shape[0]
    lys = params['layers']
    W1s = jnp.stack([p['W1'] for p in lys])
    b1s = jnp.stack([p['b1'] for p in lys])
    W2s = jnp.stack([p['W2'] for p in lys])
    b2s = jnp.stack([p['b2'] for p in lys])
    gs = jnp.stack([p['g'] for p in lys])
    bes = jnp.stack([p['be'] for p in lys])
    Wt1, bt1, Wt2 = params['Wt1'], params['bt1'].reshape(1, -1), params['Wt2']
    sc3 = jnp.stack([params['bt2'][0], params['gt'][0], params['bt'][0]]
                    ).reshape(1, 3)
    H1 = params['Ws1'].shape[1]
    H2 = params['Ws2'].shape[1]

    src3 = edge_index.reshape(2, E // 128, 128).transpose(1, 0, 2)
    ea4 = edge_attr.T.reshape(2, 8, E // 128, 128).transpose(0, 2, 1, 3)
    a_p, c_p = _make_segsum(E, T, N)(src3, ea4)
    t16 = _tc_layers(x, a_p, c_p, W1s, b1s, W2s, b2s, gs, bes, Wt1, bt1,
                     Wt2, sc3)
    sidx2 = supernode_indices.reshape(S // 128, 128)
    ws3 = params['Ws3'].reshape(-1)
    bs3p = jnp.pad(params['bs3'], (0, _L - params['bs3'].shape[0]))
    out = _make_tail(N, T, S, H1, H2)(
        t16, sidx2, params['Ws1'], params['bs1'], params['Ws2'],
        params['bs2'], ws3, bs3p)
    return out[:1].reshape(1, 1)
